# baseline (speedup 1.0000x reference)
import functools

import jax
import jax.numpy as jnp
from jax import lax
from jax.experimental import pallas as pl
from jax.experimental.pallas import tpu as pltpu
from jax.experimental.pallas import tpu_sc as plsc

_NC = 2
_NS = 16
_NW = _NC * _NS


@functools.lru_cache(maxsize=None)
def _gather_call(B, E, CH):
    per_w = B // _NW
    n_chunks = per_w // CH
    assert n_chunks >= 4 and n_chunks % 2 == 0
    mesh = plsc.VectorSubcoreMesh(core_axis_name="c", subcore_axis_name="s")

    @functools.partial(
        pl.kernel,
        mesh=mesh,
        out_type=jax.ShapeDtypeStruct((B, E), jnp.float32),
        scratch_types=[
            pltpu.VMEM((CH,), jnp.int32),
            pltpu.VMEM((CH,), jnp.int32),
            pltpu.VMEM((CH, E), jnp.float32),
            pltpu.VMEM((CH, E), jnp.float32),
            pltpu.SemaphoreType.DMA,
            pltpu.SemaphoreType.DMA,
            pltpu.SemaphoreType.DMA,
            pltpu.SemaphoreType.DMA,
        ],
        compiler_params=pltpu.CompilerParams(use_tc_tiling_on_sc=False),
    )
    def k(idx_hbm, table_hbm, out_hbm, iv0, iv1, r0, r1, g0, g1, o0, o1):
        wid = lax.axis_index("s") * _NC + lax.axis_index("c")
        base = wid * per_w
        iv = (iv0, iv1)
        rows = (r0, r1)
        g = (g0, g1)
        o = (o0, o1)

        for b in range(2):
            pltpu.sync_copy(idx_hbm.at[pl.ds(base + b * CH, CH)], iv[b])
            pltpu.async_copy(table_hbm.at[iv[b]], rows[b], g[b])

        def body(j, carry):
            for b in range(2):
                i = 2 * j + b
                off = base + i * CH
                pltpu.make_async_copy(table_hbm.at[iv[b]], rows[b], g[b]).wait()
                out_cp = pltpu.make_async_copy(
                    rows[b], out_hbm.at[pl.ds(off, CH)], o[b])
                out_cp.start()

                @pl.when(i + 2 < n_chunks)
                def _():
                    pltpu.sync_copy(
                        idx_hbm.at[pl.ds(off + 2 * CH, CH)], iv[b])
                    out_cp.wait()
                    pltpu.async_copy(table_hbm.at[iv[b]], rows[b], g[b])

            return carry

        lax.fori_loop(0, n_chunks // 2, body, 0)

        for b in range(2):
            off = base + (n_chunks - 2 + b) * CH
            pltpu.make_async_copy(
                rows[b], out_hbm.at[pl.ds(off, CH)], o[b]).wait()

    return k


@functools.lru_cache(maxsize=None)
def _xpose_call(B0, H, E, BB):
    Q = 128 // E
    grid = (H, B0 // BB)

    def body(in_ref, out_ref):
        t = in_ref[0]
        t = t.reshape(BB // Q, Q, E)
        t = jnp.transpose(t, (2, 0, 1))
        out_ref[0] = t.reshape(E, BB)

    return pl.pallas_call(
        body,
        grid=grid,
        in_specs=[
            pl.BlockSpec((1, BB // Q, 128), lambda hi, bi: (hi, bi, 0))
        ],
        out_specs=pl.BlockSpec((1, E, BB), lambda hi, bi: (hi, 0, bi)),
        out_shape=jax.ShapeDtypeStruct((H, E, B0), jnp.float32),
    )


def kernel(x, table):
    B0, H = x.shape
    E = table.shape[1]
    flat = jnp.transpose(x).reshape(B0 * H)
    g = _gather_call(B0 * H, E, 1600)(flat, table)
    gv = g.reshape(H, B0 * E // 128, 128)
    ot = _xpose_call(B0, H, E, 2048)(gv)
    return jnp.transpose(ot, (2, 0, 1))


# speedup vs baseline: 3.3979x; 3.3979x over previous
import functools

import jax
import jax.numpy as jnp
from jax import lax
from jax.experimental import pallas as pl
from jax.experimental.pallas import tpu as pltpu
from jax.experimental.pallas import tpu_sc as plsc

_NC = 2
_NS = 16
_NW = _NC * _NS


@functools.lru_cache(maxsize=None)
def _gather_call(B, E, CH):
    per_w = B // _NW
    n_chunks = per_w // CH
    assert n_chunks >= 4 and n_chunks % 2 == 0
    mesh = plsc.VectorSubcoreMesh(core_axis_name="c", subcore_axis_name="s")

    @functools.partial(
        pl.kernel,
        mesh=mesh,
        out_type=jax.ShapeDtypeStruct((B, E), jnp.float32),
        scratch_types=[
            pltpu.VMEM((CH,), jnp.int32),
            pltpu.VMEM((CH,), jnp.int32),
            pltpu.VMEM((CH, E), jnp.float32),
            pltpu.VMEM((CH, E), jnp.float32),
            pltpu.SemaphoreType.DMA,
            pltpu.SemaphoreType.DMA,
            pltpu.SemaphoreType.DMA,
            pltpu.SemaphoreType.DMA,
        ],
        compiler_params=pltpu.CompilerParams(use_tc_tiling_on_sc=False),
    )
    def k(idx_hbm, table_hbm, out_hbm, iv0, iv1, r0, r1, g0, g1, o0, o1):
        wid = lax.axis_index("s") * _NC + lax.axis_index("c")
        base = wid * per_w
        iv = (iv0, iv1)
        rows = (r0, r1)
        g = (g0, g1)
        o = (o0, o1)

        for b in range(2):
            pltpu.sync_copy(idx_hbm.at[pl.ds(base + b * CH, CH)], iv[b])
            pltpu.async_copy(table_hbm.at[iv[b]], rows[b], g[b])

        def body(j, carry):
            for b in range(2):
                i = 2 * j + b
                off = base + i * CH
                pltpu.make_async_copy(table_hbm.at[iv[b]], rows[b], g[b]).wait()
                out_cp = pltpu.make_async_copy(
                    rows[b], out_hbm.at[pl.ds(off, CH)], o[b])
                out_cp.start()

                @pl.when(i + 2 < n_chunks)
                def _():
                    pltpu.sync_copy(
                        idx_hbm.at[pl.ds(off + 2 * CH, CH)], iv[b])
                    out_cp.wait()
                    pltpu.async_copy(table_hbm.at[iv[b]], rows[b], g[b])

            return carry

        lax.fori_loop(0, n_chunks // 2, body, 0)

        for b in range(2):
            off = base + (n_chunks - 2 + b) * CH
            pltpu.make_async_copy(
                rows[b], out_hbm.at[pl.ds(off, CH)], o[b]).wait()

    return k


@functools.lru_cache(maxsize=None)
def _xpose_call(B0, H, E, BB):
    Q = 128 // E
    grid = (H, B0 // BB)

    def body(in_ref, out_ref):
        t = jnp.transpose(in_ref[0])
        out_ref[0] = jnp.concatenate(
            [t[q * E:(q + 1) * E] for q in range(Q)], axis=1)

    return pl.pallas_call(
        body,
        grid=grid,
        in_specs=[pl.BlockSpec((1, BB // Q, 128), lambda hi, bi: (hi, bi, 0))],
        out_specs=pl.BlockSpec((1, E, BB), lambda hi, bi: (hi, 0, bi)),
        out_shape=jax.ShapeDtypeStruct((H, E, B0), jnp.float32),
    )


def kernel(x, table):
    B0, H = x.shape
    E = table.shape[1]
    Q = 128 // E
    BB = 4096
    xp = jnp.transpose(x).reshape(H, B0 // BB, Q, BB // Q)
    flat = jnp.transpose(xp, (0, 1, 3, 2)).reshape(B0 * H)
    g = _gather_call(B0 * H, E, 1600)(flat, table)
    gv = g.reshape(H, B0 * E // 128, 128)
    ot = _xpose_call(B0, H, E, BB)(gv)
    return jnp.transpose(ot, (2, 0, 1))


# speedup vs baseline: 5.8120x; 1.7105x over previous
import functools

import jax
import jax.numpy as jnp
from jax import lax
from jax.experimental import pallas as pl
from jax.experimental.pallas import tpu as pltpu
from jax.experimental.pallas import tpu_sc as plsc

_NC = 2
_NS = 16
_NW = _NC * _NS


@functools.lru_cache(maxsize=None)
def _gather_call(B, E, CH):
    per_w = B // _NW
    n_chunks = per_w // CH
    assert n_chunks >= 4 and n_chunks % 2 == 0
    mesh = plsc.VectorSubcoreMesh(core_axis_name="c", subcore_axis_name="s")

    Q = 128 // E

    def _out_slice(out_hbm, off):
        blk = off // (CH * Q)
        q = (off // CH) % Q
        return out_hbm.at[pl.ds(blk * CH, CH), pl.ds(q * E, E)]

    @functools.partial(
        pl.kernel,
        mesh=mesh,
        out_type=jax.ShapeDtypeStruct((B // Q, 128), jnp.float32),
        scratch_types=[
            pltpu.VMEM((CH,), jnp.int32),
            pltpu.VMEM((CH,), jnp.int32),
            pltpu.VMEM((CH, E), jnp.float32),
            pltpu.VMEM((CH, E), jnp.float32),
            pltpu.SemaphoreType.DMA,
            pltpu.SemaphoreType.DMA,
            pltpu.SemaphoreType.DMA,
            pltpu.SemaphoreType.DMA,
        ],
        compiler_params=pltpu.CompilerParams(use_tc_tiling_on_sc=False),
    )
    def k(idx_hbm, table_hbm, out_hbm, iv0, iv1, r0, r1, g0, g1, o0, o1):
        wid = lax.axis_index("s") * _NC + lax.axis_index("c")
        base = wid * per_w
        iv = (iv0, iv1)
        rows = (r0, r1)
        g = (g0, g1)
        o = (o0, o1)

        for b in range(2):
            pltpu.sync_copy(idx_hbm.at[pl.ds(base + b * CH, CH)], iv[b])
            pltpu.async_copy(table_hbm.at[iv[b]], rows[b], g[b])

        def body(j, carry):
            for b in range(2):
                i = 2 * j + b
                off = base + i * CH
                pltpu.make_async_copy(table_hbm.at[iv[b]], rows[b], g[b]).wait()
                out_cp = pltpu.make_async_copy(
                    rows[b], _out_slice(out_hbm, off), o[b])
                out_cp.start()

                @pl.when(i + 2 < n_chunks)
                def _():
                    pltpu.sync_copy(
                        idx_hbm.at[pl.ds(off + 2 * CH, CH)], iv[b])
                    out_cp.wait()
                    pltpu.async_copy(table_hbm.at[iv[b]], rows[b], g[b])

            return carry

        lax.fori_loop(0, n_chunks // 2, body, 0)

        for b in range(2):
            off = base + (n_chunks - 2 + b) * CH
            pltpu.make_async_copy(
                rows[b], _out_slice(out_hbm, off), o[b]).wait()

    return k


@functools.lru_cache(maxsize=None)
def _xpose_call(B0, H, E, BB):
    Q = 128 // E
    grid = (H, B0 // BB)

    def body(in_ref, out_ref):
        t = jnp.transpose(in_ref[0])
        out_ref[0] = jnp.concatenate(
            [t[q * E:(q + 1) * E] for q in range(Q)], axis=1)

    return pl.pallas_call(
        body,
        grid=grid,
        in_specs=[pl.BlockSpec((1, BB // Q, 128), lambda hi, bi: (hi, bi, 0))],
        out_specs=pl.BlockSpec((1, E, BB), lambda hi, bi: (hi, 0, bi)),
        out_shape=jax.ShapeDtypeStruct((H, E, B0), jnp.float32),
    )


def kernel(x, table):
    B0, H = x.shape
    E = table.shape[1]
    flat = jnp.transpose(x).reshape(B0 * H)
    g = _gather_call(B0 * H, E, 1024)(flat, table)
    gv = g.reshape(H, B0 * E // 128, 128)
    ot = _xpose_call(B0, H, E, 4096)(gv)
    return jnp.transpose(ot, (2, 0, 1))


# speedup vs baseline: 6.6467x; 1.1436x over previous
import functools

import jax
import jax.numpy as jnp
from jax import lax
from jax.experimental import pallas as pl
from jax.experimental.pallas import tpu as pltpu
from jax.experimental.pallas import tpu_sc as plsc

_NC = 2
_NS = 16
_NW = _NC * _NS


@functools.lru_cache(maxsize=None)
def _gather_call(B, E, CH):
    per_w = B // _NW
    n_chunks = per_w // CH
    assert n_chunks >= 4 and n_chunks % 2 == 0
    mesh = plsc.VectorSubcoreMesh(core_axis_name="c", subcore_axis_name="s")

    Q = 128 // E

    def _out_slice(out_hbm, off):
        blk = off // (CH * Q)
        q = (off // CH) % Q
        return out_hbm.at[pl.ds(blk * CH, CH), pl.ds(q * E, E)]

    @functools.partial(
        pl.kernel,
        mesh=mesh,
        out_type=jax.ShapeDtypeStruct((B // Q, 128), jnp.float32),
        scratch_types=[
            pltpu.VMEM((CH,), jnp.int32),
            pltpu.VMEM((CH,), jnp.int32),
            pltpu.VMEM((CH, E), jnp.float32),
            pltpu.VMEM((CH, E), jnp.float32),
            pltpu.SemaphoreType.DMA,
            pltpu.SemaphoreType.DMA,
            pltpu.SemaphoreType.DMA,
            pltpu.SemaphoreType.DMA,
        ],
        compiler_params=pltpu.CompilerParams(use_tc_tiling_on_sc=False),
    )
    def k(idx_hbm, table_hbm, out_hbm, iv0, iv1, r0, r1, g0, g1, o0, o1):
        wid = lax.axis_index("s") * _NC + lax.axis_index("c")
        base = wid * per_w
        iv = (iv0, iv1)
        rows = (r0, r1)
        g = (g0, g1)
        o = (o0, o1)

        for b in range(2):
            pltpu.sync_copy(idx_hbm.at[pl.ds(base + b * CH, CH)], iv[b])
            pltpu.async_copy(table_hbm.at[iv[b]], rows[b], g[b])

        def body(j, carry):
            for b in range(2):
                i = 2 * j + b
                off = base + i * CH
                pltpu.make_async_copy(table_hbm.at[iv[b]], rows[b], g[b]).wait()
                out_cp = pltpu.make_async_copy(
                    rows[b], _out_slice(out_hbm, off), o[b])
                out_cp.start()

                @pl.when(i + 2 < n_chunks)
                def _():
                    pltpu.sync_copy(
                        idx_hbm.at[pl.ds(off + 2 * CH, CH)], iv[b])
                    out_cp.wait()
                    pltpu.async_copy(table_hbm.at[iv[b]], rows[b], g[b])

            return carry

        lax.fori_loop(0, n_chunks // 2, body, 0)

        for b in range(2):
            off = base + (n_chunks - 2 + b) * CH
            pltpu.make_async_copy(
                rows[b], _out_slice(out_hbm, off), o[b]).wait()

    return k


@functools.lru_cache(maxsize=None)
def _xpose_call(B0, H, E, BB):
    Q = 128 // E
    grid = (H, B0 // BB)

    def body(in_ref, out_ref):
        t = jnp.transpose(in_ref[0])
        out_ref[0] = jnp.concatenate(
            [t[q * E:(q + 1) * E] for q in range(Q)], axis=1)

    return pl.pallas_call(
        body,
        grid=grid,
        in_specs=[pl.BlockSpec((1, BB // Q, 128), lambda hi, bi: (hi, bi, 0))],
        out_specs=pl.BlockSpec((1, E, BB), lambda hi, bi: (hi, 0, bi)),
        out_shape=jax.ShapeDtypeStruct((H, E, B0), jnp.float32),
    )


def kernel(x, table):
    B0, H = x.shape
    E = table.shape[1]
    flat = jnp.transpose(x).reshape(B0 * H)
    g = _gather_call(B0 * H, E, 1024)(flat, table)
    gv = g.reshape(H, B0 * E // 128, 128)
    ot = _xpose_call(B0, H, E, 8192)(gv)
    return jnp.transpose(ot, (2, 0, 1))
